# baseline (speedup 1.0000x reference)
import functools

import jax
import jax.numpy as jnp
from jax import lax
from jax.experimental import pallas as pl
from jax.experimental.pallas import tpu as pltpu
from jax.experimental.pallas import tpu_sc as plsc

_BATCH = 4096
_SEQ = 200
_D = 64
_B = _BATCH * _SEQ
_NC = 2
_NS = 16
_NW = _NC * _NS
_IW = 128
_K = 8
_CHUNK = _K * _IW
_BPW = _B // _NW
_NCHUNK = _BPW // _CHUNK
_IROWS_PW = _BPW // _IW


@functools.partial(
    pl.kernel,
    mesh=plsc.VectorSubcoreMesh(core_axis_name="c", subcore_axis_name="s"),
    out_type=jax.ShapeDtypeStruct((_B, _D), jnp.float32),
    compiler_params=pltpu.CompilerParams(use_tc_tiling_on_sc=False),
    scratch_types=[
        pltpu.VMEM((_K, _IW), jnp.int32),
        pltpu.VMEM((_CHUNK, _D), jnp.float32),
        pltpu.SemaphoreType.DMA,
    ],
)
def _gather_kernel(table_hbm, idx_hbm, out_hbm, idx_v, rows_v, sem):
    wid = lax.axis_index("s") * _NC + lax.axis_index("c")
    irow0 = wid * _IROWS_PW
    out0 = wid * _BPW

    def body(i, carry):
        pltpu.sync_copy(idx_hbm.at[pl.ds(irow0 + i * _K, _K)], idx_v)
        copies = [
            pltpu.async_copy(
                table_hbm.at[idx_v.at[j]],
                rows_v.at[pl.ds(j * _IW, _IW)],
                sem,
            )
            for j in range(_K)
        ]
        for c in copies:
            c.wait()
        pltpu.sync_copy(rows_v, out_hbm.at[pl.ds(out0 + i * _CHUNK, _CHUNK)])
        return carry

    lax.fori_loop(0, _NCHUNK, body, 0)


def kernel(x, table):
    idx = x.reshape(_B).astype(jnp.int32).reshape(_B // _IW, _IW)
    out = _gather_kernel(table, idx)
    return out.reshape(_BATCH, _SEQ, _D)
